# baseline (speedup 1.0000x reference)
import functools

import jax
import jax.numpy as jnp
from jax import lax
from jax.experimental import pallas as pl
from jax.experimental.pallas import tpu as pltpu
from jax.experimental.pallas import tpu_sc as plsc

E = 8
EPS = 1e-09
B = 2048
D = 768
FF = 768
TB = 256
NB = B * 2 // TB + 8
ND = NB * TB
NW = 32
TPW = B // NW
NG = TPW // 16

_MESH = dict(core_axis_name="c", subcore_axis_name="s", num_cores=2,
             num_subcores=16)


def _wid():
    return lax.axis_index("s") * 2 + lax.axis_index("c")


def _lane_iota():
    return lax.broadcasted_iota(jnp.int32, (16,), 0)


def _routing_body(lg_hbm, mk_hbm, e1_hbm, e2_hbm, w1_hbm, w2_hbm, cnt_hbm,
                  lg_v, mk_v, e1_v, e2_v, w1_v, w2_v, cnt_v):
    wid = _wid()
    t0 = wid * TPW
    pltpu.sync_copy(lg_hbm.at[pl.ds(t0, TPW)], lg_v)
    pltpu.sync_copy(mk_hbm.at[pl.ds(t0, TPW)], mk_v)
    lanes = _lane_iota()
    cnt = jnp.zeros((16,), jnp.int32)
    for g in range(NG):
        rows = lanes + g * 16
        lg = [plsc.load_gather(lg_v, [rows, jnp.full((16,), e, jnp.int32)])
              for e in range(E)]
        mk = [plsc.load_gather(mk_v, [rows, jnp.full((16,), e, jnp.int32)])
              for e in range(E)]
        m = lg[0]
        for e in range(1, E):
            m = jnp.maximum(m, lg[e])
        ex = [jnp.exp(v - m) for v in lg]
        s = ex[0]
        for e in range(1, E):
            s = s + ex[e]
        gv = [jnp.where(mk[e] == 1, ex[e] / s, 0.0) for e in range(E)]
        m1 = gv[0]
        for e in range(1, E):
            m1 = jnp.maximum(m1, gv[e])
        i1 = jnp.full((16,), E, jnp.int32)
        for e in range(E - 1, -1, -1):
            i1 = jnp.where(gv[e] == m1, e, i1)
        g2 = [jnp.where(i1 == e, -1.0, gv[e]) for e in range(E)]
        m2 = g2[0]
        for e in range(1, E):
            m2 = jnp.maximum(m2, g2[e])
        i2 = jnp.full((16,), E, jnp.int32)
        for e in range(E - 1, -1, -1):
            i2 = jnp.where(g2[e] == m2, e, i2)
        denom = m1 + m2 + EPS
        sl = pl.ds(g * 16, 16)
        e1_v[sl] = i1
        e2_v[sl] = i2
        w1_v[sl] = m1 / denom
        w2_v[sl] = m2 / denom
        for e in range(E):
            pc = plsc.all_reduce_population_count(i1 == e) + \
                 plsc.all_reduce_population_count(i2 == e)
            cnt = cnt + jnp.where(lanes == e, pc, 0)
    cnt_v[0, :] = cnt
    pltpu.sync_copy(e1_v, e1_hbm.at[pl.ds(t0, TPW)])
    pltpu.sync_copy(e2_v, e2_hbm.at[pl.ds(t0, TPW)])
    pltpu.sync_copy(w1_v, w1_hbm.at[pl.ds(t0, TPW)])
    pltpu.sync_copy(w2_v, w2_hbm.at[pl.ds(t0, TPW)])
    pltpu.sync_copy(cnt_v, cnt_hbm.at[pl.ds(wid, 1)])


_routing = functools.partial(
    pl.kernel,
    _routing_body,
    out_type=(
        jax.ShapeDtypeStruct((B,), jnp.int32),
        jax.ShapeDtypeStruct((B,), jnp.int32),
        jax.ShapeDtypeStruct((B,), jnp.float32),
        jax.ShapeDtypeStruct((B,), jnp.float32),
        jax.ShapeDtypeStruct((NW, 16), jnp.int32),
    ),
    mesh=plsc.VectorSubcoreMesh(**_MESH),
    compiler_params=pltpu.CompilerParams(needs_layout_passes=False),
    scratch_types=[
        pltpu.VMEM((TPW, E), jnp.float32),
        pltpu.VMEM((TPW, E), jnp.int32),
        pltpu.VMEM((TPW,), jnp.int32),
        pltpu.VMEM((TPW,), jnp.int32),
        pltpu.VMEM((TPW,), jnp.float32),
        pltpu.VMEM((TPW,), jnp.float32),
        pltpu.VMEM((1, 16), jnp.int32),
    ],
)


def _dispatch_body(x_hbm, e1_hbm, e2_hbm, cnt_hbm,
                   xd_hbm, s1_hbm, s2_hbm, blk_hbm,
                   allcnt_v, e1_v, e2_v, s1_v, s2_v, rows_v,
                   blk_v, sem, sem2):
    wid = _wid()
    t0 = wid * TPW
    pltpu.sync_copy(cnt_hbm, allcnt_v)
    pltpu.sync_copy(e1_hbm.at[pl.ds(t0, TPW)], e1_v)
    pltpu.sync_copy(e2_hbm.at[pl.ds(t0, TPW)], e2_v)
    lanes = _lane_iota()
    n = allcnt_v[0, :]
    for w in range(1, NW):
        n = n + allcnt_v[w, :]
    nb = (n + (TB - 1)) // TB
    incl = plsc.cumsum(nb)
    base = (incl - nb) * TB
    pre = jnp.zeros((16,), jnp.int32)
    for w in range(NW):
        flag = (jnp.int32(w) < wid).astype(jnp.int32)
        pre = pre + allcnt_v[w, :] * flag
    off = base + pre
    for g in range(NG):
        sl = pl.ds(g * 16, 16)
        for src_v, dst_v in ((e1_v, s1_v), (e2_v, s2_v)):
            ev = src_v[sl]
            dest = jnp.zeros((16,), jnp.int32)
            for e in range(E):
                msk = ev == e
                cs = plsc.cumsum(msk.astype(jnp.int32))
                off_e = jnp.sum(jnp.where(lanes == e, off, 0))
                dest = jnp.where(msk, off_e + cs - 1, dest)
                pc = plsc.all_reduce_population_count(msk)
                off = off + jnp.where(lanes == e, pc, 0)
            dst_v[sl] = dest
    pltpu.sync_copy(s1_v, s1_hbm.at[pl.ds(t0, TPW)])
    pltpu.sync_copy(s2_v, s2_hbm.at[pl.ds(t0, TPW)])
    pltpu.sync_copy(x_hbm.at[pl.ds(t0, TPW)], rows_v)
    c1 = pltpu.async_copy(rows_v, xd_hbm.at[s1_v], sem)
    c2 = pltpu.async_copy(rows_v, xd_hbm.at[s2_v], sem2)
    c1.wait()
    c2.wait()

    @pl.when(wid == 0)
    def _():
        used = jnp.sum(jnp.where(lanes == E - 1, incl, 0))
        for j in range(3):
            jv = lanes + j * 16
            be = jnp.zeros((16,), jnp.int32)
            for e in range(E):
                incl_e = jnp.sum(jnp.where(lanes == e, incl, 0))
                be = be + (jv >= incl_e).astype(jnp.int32)
            blk_v[pl.ds(j * 16, 16)] = jnp.where(
                jv < used, jnp.minimum(be, E - 1), -1)
        pltpu.sync_copy(blk_v, blk_hbm)


_dispatch = functools.partial(
    pl.kernel,
    _dispatch_body,
    out_type=(
        jax.ShapeDtypeStruct((ND, D), jnp.float32),
        jax.ShapeDtypeStruct((B,), jnp.int32),
        jax.ShapeDtypeStruct((B,), jnp.int32),
        jax.ShapeDtypeStruct((48,), jnp.int32),
    ),
    mesh=plsc.VectorSubcoreMesh(**_MESH),
    compiler_params=pltpu.CompilerParams(needs_layout_passes=False),
    scratch_types=[
        pltpu.VMEM((NW, 16), jnp.int32),
        pltpu.VMEM((TPW,), jnp.int32),
        pltpu.VMEM((TPW,), jnp.int32),
        pltpu.VMEM((TPW,), jnp.int32),
        pltpu.VMEM((TPW,), jnp.int32),
        pltpu.VMEM((TPW, D), jnp.float32),
        pltpu.VMEM((48,), jnp.int32),
        pltpu.SemaphoreType.DMA,
        pltpu.SemaphoreType.DMA,
    ],
)


def _expert_body(be_ref, xd_ref, win_ref, wout_ref, b_ref, o_ref,
                 win_bf, wout_bf):
    i = pl.program_id(0)
    be = be_ref[i]
    prev = be_ref[jnp.maximum(i - 1, 0)]

    @pl.when(be >= 0)
    def _():
        @pl.when((i == 0) | (be != prev))
        def _():
            win_bf[...] = win_ref[be].astype(jnp.bfloat16)
            wout_bf[...] = wout_ref[be].astype(jnp.bfloat16)

        xb = xd_ref[...].astype(jnp.bfloat16)
        h = jnp.dot(xb, win_bf[...], preferred_element_type=jnp.float32)
        h = h * 0.5 * (1.0 + lax.erf(h * 0.7071067811865476))
        y = jnp.dot(h.astype(jnp.bfloat16), wout_bf[...],
                    preferred_element_type=jnp.float32)
        y = y + b_ref[be]
        o_ref[...] = y


def _expert(blk_e, xd, W_in, W_out, b_out):
    grid_spec = pltpu.PrefetchScalarGridSpec(
        num_scalar_prefetch=1,
        grid=(NB,),
        in_specs=[
            pl.BlockSpec((TB, D), lambda i, be: (i, 0)),
            pl.BlockSpec((E, D, FF), lambda i, be: (0, 0, 0)),
            pl.BlockSpec((E, FF, D), lambda i, be: (0, 0, 0)),
            pl.BlockSpec((E, 1, D), lambda i, be: (0, 0, 0)),
        ],
        out_specs=pl.BlockSpec((TB, D), lambda i, be: (i, 0)),
        scratch_shapes=[
            pltpu.VMEM((D, FF), jnp.bfloat16),
            pltpu.VMEM((FF, D), jnp.bfloat16),
        ],
    )
    return pl.pallas_call(
        _expert_body,
        grid_spec=grid_spec,
        out_shape=jax.ShapeDtypeStruct((ND, D), jnp.float32),
        compiler_params=pltpu.CompilerParams(
            dimension_semantics=("arbitrary",),
        ),
    )(blk_e, xd, W_in, W_out, b_out)


def _cast_body(x_ref, o_ref):
    o_ref[...] = x_ref[...].astype(jnp.bfloat16)


def _cast_bf16(x):
    return pl.pallas_call(
        _cast_body,
        grid=(4,),
        in_specs=[pl.BlockSpec((B // 4, D), lambda i: (i, 0))],
        out_specs=pl.BlockSpec((B // 4, D), lambda i: (i, 0)),
        out_shape=jax.ShapeDtypeStruct((B, D), jnp.bfloat16),
    )(x)


def _combine_body(yd_hbm, s1_hbm, s2_hbm, w1_hbm, w2_hbm, out_hbm,
                  s1_v, s2_v, w1_v, w2_v, r1_v, r2_v, sem, sem2):
    wid = _wid()
    t0 = wid * TPW
    pltpu.sync_copy(s1_hbm.at[pl.ds(t0, TPW)], s1_v)
    pltpu.sync_copy(s2_hbm.at[pl.ds(t0, TPW)], s2_v)
    pltpu.sync_copy(w1_hbm.at[pl.ds(t0, TPW)], w1_v)
    pltpu.sync_copy(w2_hbm.at[pl.ds(t0, TPW)], w2_v)
    c1 = pltpu.async_copy(yd_hbm.at[s1_v], r1_v, sem)
    c2 = pltpu.async_copy(yd_hbm.at[s2_v], r2_v, sem2)
    c1.wait()
    c2.wait()
    lanes = _lane_iota()

    def row(j, carry):
        grp = j // 16
        lane = j - grp * 16
        sel = lanes == lane
        w1g = w1_v[pl.ds(grp * 16, 16)]
        w2g = w2_v[pl.ds(grp * 16, 16)]
        w1j = jnp.sum(jnp.where(sel, w1g, 0.0))
        w2j = jnp.sum(jnp.where(sel, w2g, 0.0))
        for c in range(D // 16):
            sl = (j, pl.ds(c * 16, 16))
            r1_v[sl] = r1_v[sl] * w1j + r2_v[sl] * w2j
        return carry

    lax.fori_loop(0, TPW, row, 0)
    pltpu.sync_copy(r1_v, out_hbm.at[pl.ds(t0, TPW)])


_combine = functools.partial(
    pl.kernel,
    _combine_body,
    out_type=jax.ShapeDtypeStruct((B, D), jnp.float32),
    mesh=plsc.VectorSubcoreMesh(**_MESH),
    compiler_params=pltpu.CompilerParams(needs_layout_passes=False),
    scratch_types=[
        pltpu.VMEM((TPW,), jnp.int32),
        pltpu.VMEM((TPW,), jnp.int32),
        pltpu.VMEM((TPW,), jnp.float32),
        pltpu.VMEM((TPW,), jnp.float32),
        pltpu.VMEM((TPW, D), jnp.float32),
        pltpu.VMEM((TPW, D), jnp.float32),
        pltpu.SemaphoreType.DMA,
        pltpu.SemaphoreType.DMA,
    ],
)


def kernel(cycle_curve_data, logits, moe_masks, W_in, W_out, b_out):
    x = cycle_curve_data.reshape(B, D)
    e1, e2, w1, w2, cnt = _routing()(logits, moe_masks)
    xd, s1, s2, blk_e = _dispatch()(x, e1, e2, cnt)
    yd = _expert(blk_e[:NB], xd, W_in, W_out, b_out.reshape(E, 1, D))
    out = _combine()(yd, s1, s2, w1, w2)
    return _cast_bf16(out).reshape(B, 1, D)


# speedup vs baseline: 1.0162x; 1.0162x over previous
import functools

import jax
import jax.numpy as jnp
from jax import lax
from jax.experimental import pallas as pl
from jax.experimental.pallas import tpu as pltpu
from jax.experimental.pallas import tpu_sc as plsc

E = 8
EPS = 1e-09
B = 2048
D = 768
FF = 768
TB = 512
NB = B * 2 // TB + 8
ND = NB * TB
NW = 32
TPW = B // NW
NG = TPW // 16

_MESH = dict(core_axis_name="c", subcore_axis_name="s", num_cores=2,
             num_subcores=16)


def _wid():
    return lax.axis_index("s") * 2 + lax.axis_index("c")


def _lane_iota():
    return lax.broadcasted_iota(jnp.int32, (16,), 0)


def _routing_body(lg_hbm, mk_hbm, e1_hbm, e2_hbm, w1_hbm, w2_hbm, cnt_hbm,
                  lg_v, mk_v, e1_v, e2_v, w1_v, w2_v, cnt_v):
    wid = _wid()
    t0 = wid * TPW
    pltpu.sync_copy(lg_hbm.at[pl.ds(t0, TPW)], lg_v)
    pltpu.sync_copy(mk_hbm.at[pl.ds(t0, TPW)], mk_v)
    lanes = _lane_iota()
    cnt = jnp.zeros((16,), jnp.int32)
    for g in range(NG):
        rows = lanes + g * 16
        lg = [plsc.load_gather(lg_v, [rows, jnp.full((16,), e, jnp.int32)])
              for e in range(E)]
        mk = [plsc.load_gather(mk_v, [rows, jnp.full((16,), e, jnp.int32)])
              for e in range(E)]
        m = lg[0]
        for e in range(1, E):
            m = jnp.maximum(m, lg[e])
        ex = [jnp.exp(v - m) for v in lg]
        s = ex[0]
        for e in range(1, E):
            s = s + ex[e]
        gv = [jnp.where(mk[e] == 1, ex[e] / s, 0.0) for e in range(E)]
        m1 = gv[0]
        for e in range(1, E):
            m1 = jnp.maximum(m1, gv[e])
        i1 = jnp.full((16,), E, jnp.int32)
        for e in range(E - 1, -1, -1):
            i1 = jnp.where(gv[e] == m1, e, i1)
        g2 = [jnp.where(i1 == e, -1.0, gv[e]) for e in range(E)]
        m2 = g2[0]
        for e in range(1, E):
            m2 = jnp.maximum(m2, g2[e])
        i2 = jnp.full((16,), E, jnp.int32)
        for e in range(E - 1, -1, -1):
            i2 = jnp.where(g2[e] == m2, e, i2)
        denom = m1 + m2 + EPS
        sl = pl.ds(g * 16, 16)
        e1_v[sl] = i1
        e2_v[sl] = i2
        w1_v[sl] = m1 / denom
        w2_v[sl] = m2 / denom
        for e in range(E):
            pc = plsc.all_reduce_population_count(i1 == e) + \
                 plsc.all_reduce_population_count(i2 == e)
            cnt = cnt + jnp.where(lanes == e, pc, 0)
    cnt_v[0, :] = cnt
    pltpu.sync_copy(e1_v, e1_hbm.at[pl.ds(t0, TPW)])
    pltpu.sync_copy(e2_v, e2_hbm.at[pl.ds(t0, TPW)])
    pltpu.sync_copy(w1_v, w1_hbm.at[pl.ds(t0, TPW)])
    pltpu.sync_copy(w2_v, w2_hbm.at[pl.ds(t0, TPW)])
    pltpu.sync_copy(cnt_v, cnt_hbm.at[pl.ds(wid, 1)])


_routing = functools.partial(
    pl.kernel,
    _routing_body,
    out_type=(
        jax.ShapeDtypeStruct((B,), jnp.int32),
        jax.ShapeDtypeStruct((B,), jnp.int32),
        jax.ShapeDtypeStruct((B,), jnp.float32),
        jax.ShapeDtypeStruct((B,), jnp.float32),
        jax.ShapeDtypeStruct((NW, 16), jnp.int32),
    ),
    mesh=plsc.VectorSubcoreMesh(**_MESH),
    compiler_params=pltpu.CompilerParams(needs_layout_passes=False),
    scratch_types=[
        pltpu.VMEM((TPW, E), jnp.float32),
        pltpu.VMEM((TPW, E), jnp.int32),
        pltpu.VMEM((TPW,), jnp.int32),
        pltpu.VMEM((TPW,), jnp.int32),
        pltpu.VMEM((TPW,), jnp.float32),
        pltpu.VMEM((TPW,), jnp.float32),
        pltpu.VMEM((1, 16), jnp.int32),
    ],
)


def _dispatch_body(x_hbm, e1_hbm, e2_hbm, cnt_hbm,
                   xd_hbm, s1_hbm, s2_hbm, blk_hbm,
                   allcnt_v, e1_v, e2_v, s1_v, s2_v, rows_v,
                   blk_v, sem, sem2):
    wid = _wid()
    t0 = wid * TPW
    pltpu.sync_copy(cnt_hbm, allcnt_v)
    pltpu.sync_copy(e1_hbm.at[pl.ds(t0, TPW)], e1_v)
    pltpu.sync_copy(e2_hbm.at[pl.ds(t0, TPW)], e2_v)
    lanes = _lane_iota()
    n = allcnt_v[0, :]
    for w in range(1, NW):
        n = n + allcnt_v[w, :]
    nb = (n + (TB - 1)) // TB
    incl = plsc.cumsum(nb)
    base = (incl - nb) * TB
    pre = jnp.zeros((16,), jnp.int32)
    for w in range(NW):
        flag = (jnp.int32(w) < wid).astype(jnp.int32)
        pre = pre + allcnt_v[w, :] * flag
    off = base + pre
    for g in range(NG):
        sl = pl.ds(g * 16, 16)
        for src_v, dst_v in ((e1_v, s1_v), (e2_v, s2_v)):
            ev = src_v[sl]
            dest = jnp.zeros((16,), jnp.int32)
            for e in range(E):
                msk = ev == e
                cs = plsc.cumsum(msk.astype(jnp.int32))
                off_e = jnp.sum(jnp.where(lanes == e, off, 0))
                dest = jnp.where(msk, off_e + cs - 1, dest)
                pc = plsc.all_reduce_population_count(msk)
                off = off + jnp.where(lanes == e, pc, 0)
            dst_v[sl] = dest
    pltpu.sync_copy(s1_v, s1_hbm.at[pl.ds(t0, TPW)])
    pltpu.sync_copy(s2_v, s2_hbm.at[pl.ds(t0, TPW)])
    pltpu.sync_copy(x_hbm.at[pl.ds(t0, TPW)], rows_v)
    c1 = pltpu.async_copy(rows_v, xd_hbm.at[s1_v], sem)
    c2 = pltpu.async_copy(rows_v, xd_hbm.at[s2_v], sem2)
    c1.wait()
    c2.wait()

    @pl.when(wid == 0)
    def _():
        used = jnp.sum(jnp.where(lanes == E - 1, incl, 0))
        for j in range(3):
            jv = lanes + j * 16
            be = jnp.zeros((16,), jnp.int32)
            for e in range(E):
                incl_e = jnp.sum(jnp.where(lanes == e, incl, 0))
                be = be + (jv >= incl_e).astype(jnp.int32)
            blk_v[pl.ds(j * 16, 16)] = jnp.where(
                jv < used, jnp.minimum(be, E - 1), -1)
        pltpu.sync_copy(blk_v, blk_hbm)


_dispatch = functools.partial(
    pl.kernel,
    _dispatch_body,
    out_type=(
        jax.ShapeDtypeStruct((ND, D), jnp.float32),
        jax.ShapeDtypeStruct((B,), jnp.int32),
        jax.ShapeDtypeStruct((B,), jnp.int32),
        jax.ShapeDtypeStruct((48,), jnp.int32),
    ),
    mesh=plsc.VectorSubcoreMesh(**_MESH),
    compiler_params=pltpu.CompilerParams(needs_layout_passes=False),
    scratch_types=[
        pltpu.VMEM((NW, 16), jnp.int32),
        pltpu.VMEM((TPW,), jnp.int32),
        pltpu.VMEM((TPW,), jnp.int32),
        pltpu.VMEM((TPW,), jnp.int32),
        pltpu.VMEM((TPW,), jnp.int32),
        pltpu.VMEM((TPW, D), jnp.float32),
        pltpu.VMEM((48,), jnp.int32),
        pltpu.SemaphoreType.DMA,
        pltpu.SemaphoreType.DMA,
    ],
)


def _expert_body(be_ref, xd_ref, win_ref, wout_ref, b_ref, o_ref,
                 win_bf, wout_bf):
    i = pl.program_id(0)
    be = be_ref[i]
    prev = be_ref[jnp.maximum(i - 1, 0)]

    @pl.when(be >= 0)
    def _():
        @pl.when((i == 0) | (be != prev))
        def _():
            win_bf[...] = win_ref[be].astype(jnp.bfloat16)
            wout_bf[...] = wout_ref[be].astype(jnp.bfloat16)

        xb = xd_ref[...].astype(jnp.bfloat16)
        h = jnp.dot(xb, win_bf[...], preferred_element_type=jnp.float32)
        h = h * 0.5 * (1.0 + lax.erf(h * 0.7071067811865476))
        y = jnp.dot(h.astype(jnp.bfloat16), wout_bf[...],
                    preferred_element_type=jnp.float32)
        y = y + b_ref[be]
        o_ref[...] = y


def _expert(blk_e, xd, W_in, W_out, b_out):
    grid_spec = pltpu.PrefetchScalarGridSpec(
        num_scalar_prefetch=1,
        grid=(NB,),
        in_specs=[
            pl.BlockSpec((TB, D), lambda i, be: (i, 0)),
            pl.BlockSpec((E, D, FF), lambda i, be: (0, 0, 0)),
            pl.BlockSpec((E, FF, D), lambda i, be: (0, 0, 0)),
            pl.BlockSpec((E, 1, D), lambda i, be: (0, 0, 0)),
        ],
        out_specs=pl.BlockSpec((TB, D), lambda i, be: (i, 0)),
        scratch_shapes=[
            pltpu.VMEM((D, FF), jnp.bfloat16),
            pltpu.VMEM((FF, D), jnp.bfloat16),
        ],
    )
    return pl.pallas_call(
        _expert_body,
        grid_spec=grid_spec,
        out_shape=jax.ShapeDtypeStruct((ND, D), jnp.float32),
        compiler_params=pltpu.CompilerParams(
            dimension_semantics=("arbitrary",),
        ),
    )(blk_e, xd, W_in, W_out, b_out)


def _cast_body(x_ref, o_ref):
    o_ref[...] = x_ref[...].astype(jnp.bfloat16)


def _cast_bf16(x):
    return pl.pallas_call(
        _cast_body,
        grid=(4,),
        in_specs=[pl.BlockSpec((B // 4, D), lambda i: (i, 0))],
        out_specs=pl.BlockSpec((B // 4, D), lambda i: (i, 0)),
        out_shape=jax.ShapeDtypeStruct((B, D), jnp.bfloat16),
    )(x)


def _combine_body(yd_hbm, s1_hbm, s2_hbm, w1_hbm, w2_hbm, out_hbm,
                  s1_v, s2_v, w1_v, w2_v, r1_v, r2_v, sem, sem2):
    wid = _wid()
    t0 = wid * TPW
    pltpu.sync_copy(s1_hbm.at[pl.ds(t0, TPW)], s1_v)
    pltpu.sync_copy(s2_hbm.at[pl.ds(t0, TPW)], s2_v)
    pltpu.sync_copy(w1_hbm.at[pl.ds(t0, TPW)], w1_v)
    pltpu.sync_copy(w2_hbm.at[pl.ds(t0, TPW)], w2_v)
    c1 = pltpu.async_copy(yd_hbm.at[s1_v], r1_v, sem)
    c2 = pltpu.async_copy(yd_hbm.at[s2_v], r2_v, sem2)
    c1.wait()
    c2.wait()
    lanes = _lane_iota()

    def row(j, carry):
        grp = j // 16
        lane = j - grp * 16
        sel = lanes == lane
        w1g = w1_v[pl.ds(grp * 16, 16)]
        w2g = w2_v[pl.ds(grp * 16, 16)]
        w1j = jnp.sum(jnp.where(sel, w1g, 0.0))
        w2j = jnp.sum(jnp.where(sel, w2g, 0.0))
        for c in range(D // 16):
            sl = (j, pl.ds(c * 16, 16))
            r1_v[sl] = r1_v[sl] * w1j + r2_v[sl] * w2j
        return carry

    lax.fori_loop(0, TPW, row, 0)
    pltpu.sync_copy(r1_v, out_hbm.at[pl.ds(t0, TPW)])


_combine = functools.partial(
    pl.kernel,
    _combine_body,
    out_type=jax.ShapeDtypeStruct((B, D), jnp.float32),
    mesh=plsc.VectorSubcoreMesh(**_MESH),
    compiler_params=pltpu.CompilerParams(needs_layout_passes=False),
    scratch_types=[
        pltpu.VMEM((TPW,), jnp.int32),
        pltpu.VMEM((TPW,), jnp.int32),
        pltpu.VMEM((TPW,), jnp.float32),
        pltpu.VMEM((TPW,), jnp.float32),
        pltpu.VMEM((TPW, D), jnp.float32),
        pltpu.VMEM((TPW, D), jnp.float32),
        pltpu.SemaphoreType.DMA,
        pltpu.SemaphoreType.DMA,
    ],
)


def kernel(cycle_curve_data, logits, moe_masks, W_in, W_out, b_out):
    x = cycle_curve_data.reshape(B, D)
    e1, e2, w1, w2, cnt = _routing()(logits, moe_masks)
    xd, s1, s2, blk_e = _dispatch()(x, e1, e2, cnt)
    yd = _expert(blk_e[:NB], xd, W_in, W_out, b_out.reshape(E, 1, D))
    out = _combine()(yd, s1, s2, w1, w2)
    return _cast_bf16(out).reshape(B, 1, D)
